# baseline (speedup 1.0000x reference)
import functools

import jax
import jax.numpy as jnp
from jax import lax
from jax.experimental import pallas as pl
from jax.experimental.pallas import tpu as pltpu
from jax.experimental.pallas import tpu_sc as plsc

N = 10000
E = 320000
D_IN = 128
HEADS = 8
HID = 16
DHH = HEADS * HID
REL_IN = 64
REL_HID = 16
NEG = 0.2

NT = 16
ROWS_PT = 632
NPAD = NT * ROWS_PT
EPT = E // NT
CH = 32
NCH = EPT // CH
HP = 16
S_ROWS = 1280
SPT = S_ROWS // NT


def _leaky(x):
    return jnp.where(x >= 0, x, NEG * x)


def _dense_body(f_ref, wn_ref, ws_ref, bs_ref, bd_ref,
                h_ref, hs_ref, es_ref, ed_ref):
    f = f_ref[...]
    h = f @ wn_ref[...]
    hs = h @ ws_ref[...]
    h_ref[...] = h
    hs_ref[...] = hs
    es_ref[...] = hs @ bs_ref[...]
    ed_ref[...] = h @ bd_ref[...]


def _dense(feat, W_node, W_src, Bsrc, Bdst):
    nb = 10
    return pl.pallas_call(
        _dense_body,
        out_shape=(
            jax.ShapeDtypeStruct((N, DHH), jnp.float32),
            jax.ShapeDtypeStruct((N, DHH), jnp.float32),
            jax.ShapeDtypeStruct((N, DHH), jnp.float32),
            jax.ShapeDtypeStruct((N, DHH), jnp.float32),
        ),
        grid=(nb,),
        in_specs=[
            pl.BlockSpec((N // nb, D_IN), lambda i: (i, 0)),
            pl.BlockSpec((D_IN, DHH), lambda i: (0, 0)),
            pl.BlockSpec((DHH, DHH), lambda i: (0, 0)),
            pl.BlockSpec((DHH, DHH), lambda i: (0, 0)),
            pl.BlockSpec((DHH, DHH), lambda i: (0, 0)),
        ],
        out_specs=(
            pl.BlockSpec((N // nb, DHH), lambda i: (i, 0)),
            pl.BlockSpec((N // nb, DHH), lambda i: (i, 0)),
            pl.BlockSpec((N // nb, DHH), lambda i: (i, 0)),
            pl.BlockSpec((N // nb, DHH), lambda i: (i, 0)),
        ),
    )(feat, W_node, W_src, Bsrc, Bdst)


def _rel_body(re_ref, wr_ref, wp_ref, bp_ref, rw_ref, rp_ref):
    re = re_ref[...]
    rw_ref[...] = re @ wr_ref[...]
    rp_ref[...] = re @ wp_ref[...] + bp_ref[...]


def _rel(rel_emb, W_rel, W_prop, b_prop):
    return pl.pallas_call(
        _rel_body,
        out_shape=(
            jax.ShapeDtypeStruct((1, HEADS * 2 * HID), jnp.float32),
            jax.ShapeDtypeStruct((1, HEADS * REL_HID), jnp.float32),
        ),
    )(rel_emb.reshape(1, REL_IN), W_rel, W_prop, b_prop.reshape(1, -1))


_mesh = plsc.VectorSubcoreMesh(core_axis_name="c", subcore_axis_name="s")


@functools.partial(
    pl.kernel,
    out_type=(
        jax.ShapeDtypeStruct((2 * NPAD, DHH), jnp.float32),
        jax.ShapeDtypeStruct((2 * S_ROWS, DHH), jnp.float32),
    ),
    mesh=_mesh,
    scratch_types=(
        [pltpu.VMEM((CH,), jnp.int32)] * 12
        + [pltpu.VMEM((CH, DHH), jnp.float32)] * 9
        + [
            pltpu.VMEM((HP,), jnp.float32),
            pltpu.VMEM_SHARED((NPAD, DHH), jnp.float32),
            pltpu.VMEM_SHARED((S_ROWS, DHH), jnp.float32),
        ]
        + [pltpu.SemaphoreType.DMA] * 9
    ),
)
def _sc_edge(es_hbm, ed_hbm, hs_hbm, src_hbm, dstg_hbm, dstl_hbm, shift_hbm,
             z_hbm, out_hbm, s_out_hbm,
             src_v0, src_v1, src_v2, dstg_v0, dstg_v1, dstg_v2,
             dstl_v0, dstl_v1, dstl_v2, sidx_v0, sidx_v1, sidx_v2,
             esx_v0, esx_v1, esx_v2, ed_v0, ed_v1, ed_v2,
             hs_v0, hs_v1, hs_v2, shift_v, out_sh, s_sh,
             ixsem0, ixsem1, ixsem2, gsem0, gsem1, gsem2,
             ssem0, ssem1, ssem2):
    cid = lax.axis_index("c")
    sid = lax.axis_index("s")

    srcs = (src_v0, src_v1, src_v2)
    dstgs = (dstg_v0, dstg_v1, dstg_v2)
    dstls = (dstl_v0, dstl_v1, dstl_v2)
    sidxs = (sidx_v0, sidx_v1, sidx_v2)
    esxs = (esx_v0, esx_v1, esx_v2)
    eds = (ed_v0, ed_v1, ed_v2)
    hss = (hs_v0, hs_v1, hs_v2)
    ixsems = (ixsem0, ixsem1, ixsem2)
    gsems = (gsem0, gsem1, gsem2)
    ssems = (ssem0, ssem1, ssem2)

    pltpu.sync_copy(z_hbm, out_sh.at[pl.ds(sid * ROWS_PT, ROWS_PT)])
    pltpu.sync_copy(z_hbm.at[pl.ds(0, SPT)], s_sh.at[pl.ds(sid * SPT, SPT)])
    pltpu.sync_copy(shift_hbm.at[pl.ds(cid * HP, HP)], shift_v)
    plsc.subcore_barrier()

    ebase = cid * E + sid * EPT
    sh = shift_v[...]
    zv = jnp.zeros((HP,), jnp.float32)

    def issue_idx(ci, b):
        base = ebase + ci * CH
        sem = ixsems[b]
        return (pltpu.async_copy(src_hbm.at[pl.ds(base, CH)], srcs[b], sem),
                pltpu.async_copy(dstg_hbm.at[pl.ds(base, CH)], dstgs[b], sem),
                pltpu.async_copy(dstl_hbm.at[pl.ds(base, CH)], dstls[b], sem))

    def issue_gather(b):
        sem = gsems[b]
        return (pltpu.async_copy(es_hbm.at[srcs[b]], esxs[b], sem),
                pltpu.async_copy(ed_hbm.at[dstgs[b]], eds[b], sem),
                pltpu.async_copy(hs_hbm.at[srcs[b]], hss[b], sem))

    def compute(b, g):
        dstl_v, sidx_v, esx_v, ed_v, hs_v = (
            dstls[b], sidxs[b], esxs[b], eds[b], hss[b])
        g[0].wait()
        g[1].wait()
        g[2].wait()

        def group(gi, carry2):
            dvec = dstl_v[pl.ds(gi * HP, HP)]
            sidx_v[pl.ds(gi * HP, HP)] = lax.shift_right_logical(dvec, 3)
            for k in range(HP):
                e = gi * HP + k
                x = esx_v[e, pl.ds(0, HP)] + ed_v[e, pl.ds(0, HP)]
                x = jnp.where(x >= 0, x, NEG * x)
                ex = jnp.exp(x - sh)
                d = dvec[k]
                for slot in range(8):
                    esx_v[e, pl.ds(slot * HP, HP)] = zv
                esx_v[e, pl.ds((d & 7) * HP, HP)] = ex
                for h in range(HEADS):
                    av = jnp.full((HID,), ex[h], jnp.float32)
                    col = h * HID
                    hs_v[e, pl.ds(col, HID)] = hs_v[e, pl.ds(col, HID)] * av
            return carry2

        lax.fori_loop(0, CH // HP, group, 0)
        sem = ssems[b]
        return (pltpu.async_copy(esx_v, s_sh.at[sidx_v], sem, add=True),
                pltpu.async_copy(hs_v, out_sh.at[dstl_v], sem, add=True))

    def fetch(ci, b):
        for h in issue_idx(ci, b):
            h.wait()
        return issue_gather(b)

    def six(i, carry):
        c = 6 * i
        ix0 = issue_idx(c, 0)
        ix1 = issue_idx(c + 1, 1)
        ix2 = issue_idx(c + 2, 2)
        for h in ix0:
            h.wait()
        g0 = issue_gather(0)
        for h in ix1:
            h.wait()
        g1 = issue_gather(1)
        for h in ix2:
            h.wait()
        g2 = issue_gather(2)
        s0 = compute(0, g0)
        s1 = compute(1, g1)
        for h in s0:
            h.wait()
        g3 = fetch(c + 3, 0)
        s2 = compute(2, g2)
        for h in s1:
            h.wait()
        g4 = fetch(c + 4, 1)
        s3 = compute(0, g3)
        for h in s2:
            h.wait()
        g5 = fetch(c + 5, 2)
        s4 = compute(1, g4)
        s5 = compute(2, g5)
        for h in s3 + s4 + s5:
            h.wait()
        return carry

    lax.fori_loop(0, NCH // 6, six, 0)

    st = compute(0, fetch(NCH - 1, 0))
    for h in st:
        h.wait()

    plsc.subcore_barrier()
    pltpu.sync_copy(out_sh.at[pl.ds(sid * ROWS_PT, ROWS_PT)],
                    out_hbm.at[pl.ds(cid * NPAD + sid * ROWS_PT, ROWS_PT)])
    pltpu.sync_copy(s_sh.at[pl.ds(sid * SPT, SPT)],
                    s_out_hbm.at[pl.ds(cid * S_ROWS + sid * SPT, SPT)])


def _post_body(om1_ref, om2_ref, s1_ref, s2_ref, f1_ref, f2_ref,
               wres_ref, bres_ref, rw_ref, eh16_ref, eh8_ref,
               bc1_ref, bc2_ref, o1_ref, o2_ref):
    eh16 = eh16_ref[...]
    eh8 = eh8_ref[...]
    alpha = 1.0 / (1.0 + jnp.exp(-rw_ref[0, 0]))

    inv1 = 1.0 / (s1_ref[...] + 1e-16)
    out1 = jnp.maximum(om1_ref[...] * (inv1 @ eh8), 0.0)
    inv2 = 1.0 / (s2_ref[...] + 1e-16)
    out2 = jnp.maximum(om2_ref[...] * (inv2 @ eh8), 0.0)

    res1 = f1_ref[...] @ wres_ref[...] + bres_ref[...]
    res2 = f2_ref[...] @ wres_ref[...] + bres_ref[...]
    out1 = out1 * alpha + res1 * (1.0 - alpha)
    out2 = out2 * alpha + res2 * (1.0 - alpha)

    def cross(bc):
        t1 = _leaky(out1 @ bc)
        t2 = _leaky(out2 @ bc)
        m = jnp.maximum(t1, t2)
        w1 = jnp.exp(t1 - m)
        w2 = jnp.exp(t2 - m)
        sw = w1 + w2
        return out1 * ((w1 / sw) @ eh8) + out2 * ((w2 / sw) @ eh8)

    o1_ref[...] = cross(bc1_ref[...])
    o2_ref[...] = cross(bc2_ref[...])


def _post(om1, om2, s1, s2, feat1, feat2, W_res, b_res, res_w, Eh16, Eh8,
          Bc1, Bc2):
    nb = 10
    blk = N // nb
    return pl.pallas_call(
        _post_body,
        out_shape=(
            jax.ShapeDtypeStruct((N, DHH), jnp.float32),
            jax.ShapeDtypeStruct((N, DHH), jnp.float32),
        ),
        grid=(nb,),
        in_specs=[
            pl.BlockSpec((blk, DHH), lambda i: (i, 0)),
            pl.BlockSpec((blk, DHH), lambda i: (i, 0)),
            pl.BlockSpec((blk, HEADS), lambda i: (i, 0)),
            pl.BlockSpec((blk, HEADS), lambda i: (i, 0)),
            pl.BlockSpec((blk, D_IN), lambda i: (i, 0)),
            pl.BlockSpec((blk, D_IN), lambda i: (i, 0)),
            pl.BlockSpec((D_IN, DHH), lambda i: (0, 0)),
            pl.BlockSpec((1, DHH), lambda i: (0, 0)),
            pl.BlockSpec((1, 1), lambda i: (0, 0)),
            pl.BlockSpec((HP, DHH), lambda i: (0, 0)),
            pl.BlockSpec((HEADS, DHH), lambda i: (0, 0)),
            pl.BlockSpec((DHH, HEADS), lambda i: (0, 0)),
            pl.BlockSpec((DHH, HEADS), lambda i: (0, 0)),
        ],
        out_specs=(
            pl.BlockSpec((blk, DHH), lambda i: (i, 0)),
            pl.BlockSpec((blk, DHH), lambda i: (i, 0)),
        ),
    )(om1, om2, s1, s2, feat1, feat2, W_res, b_res.reshape(1, DHH),
      res_w.reshape(1, 1), Eh16, Eh8, Bc1, Bc2)


def kernel(feat1, feat2, rel_emb1, rel_emb2, edge_index1, edge_index2,
           W_node, W_src1, W_src2, W_rel1, W_rel2,
           W_res, b_res, res_w, cross_w1, cross_w2,
           W_prop1, W_prop2, b_prop1, b_prop2):
    rw1, rp1 = _rel(rel_emb1, W_rel1, W_prop1, b_prop1)
    rw2, rp2 = _rel(rel_emb2, W_rel2, W_prop2, b_prop2)

    m8 = jnp.repeat(jnp.eye(HEADS, dtype=jnp.float32), HID, axis=0)
    m128 = jnp.pad(m8, ((0, 0), (0, DHH - HEADS)))
    rwa = rw1.reshape(HEADS, 2 * HID)
    rwb = rw2.reshape(HEADS, 2 * HID)
    Bdst1 = rwa[:, :HID].reshape(DHH, 1) * m128
    Bsrc1 = rwa[:, HID:].reshape(DHH, 1) * m128
    Bdst2 = rwb[:, :HID].reshape(DHH, 1) * m128
    Bsrc2 = rwb[:, HID:].reshape(DHH, 1) * m128

    h1, hs1, es1, ed1 = _dense(feat1, W_node, W_src1, Bsrc1, Bdst1)
    h2, hs2, es2, ed2 = _dense(feat2, W_node, W_src2, Bsrc2, Bdst2)

    def shift_of(es, ed):
        c = jnp.maximum(jnp.max(es[:, :HEADS], axis=0)
                        + jnp.max(ed[:, :HEADS], axis=0), 0.0)
        return jnp.pad(c, (0, HP - HEADS))

    shift = jnp.concatenate([shift_of(es1, ed1), shift_of(es2, ed2)])

    es_all = jnp.concatenate([es1, es2])
    ed_all = jnp.concatenate([ed1, ed2])
    hs_all = jnp.concatenate([hs1, hs2])
    src_g = jnp.concatenate([edge_index1[0], edge_index2[0] + N])
    dst_g = jnp.concatenate([edge_index1[1], edge_index2[1] + N])
    dst_l = jnp.concatenate([edge_index1[1], edge_index2[1]])
    z = jnp.zeros((ROWS_PT, DHH), jnp.float32)

    out_msg, s_pack = _sc_edge(es_all, ed_all, hs_all, src_g, dst_g, dst_l,
                               shift, z)

    Eh8 = jnp.repeat(jnp.eye(HEADS, dtype=jnp.float32), HID, axis=1)
    Eh16 = jnp.pad(Eh8, ((0, HP - HEADS), (0, 0)))
    Bc1 = cross_w1.reshape(DHH, 1) * m8
    Bc2 = cross_w2.reshape(DHH, 1) * m8

    s_un = s_pack.reshape(2, S_ROWS * 8, HP)[:, :N, :HEADS]
    o1, o2 = _post(out_msg[:N], out_msg[NPAD:NPAD + N],
                   s_un[0], s_un[1],
                   feat1, feat2, W_res, b_res, res_w, Eh16, Eh8, Bc1, Bc2)

    return (o1, o2, rp1.reshape(-1), rp2.reshape(-1),
            h1.reshape(N, HEADS, HID), h2.reshape(N, HEADS, HID))


# speedup vs baseline: 1.0503x; 1.0503x over previous
import functools

import jax
import jax.numpy as jnp
from jax import lax
from jax.experimental import pallas as pl
from jax.experimental.pallas import tpu as pltpu
from jax.experimental.pallas import tpu_sc as plsc

N = 10000
E = 320000
D_IN = 128
HEADS = 8
HID = 16
DHH = HEADS * HID
REL_IN = 64
REL_HID = 16
NEG = 0.2

NT = 16
ROWS_PT = 632
NPAD = NT * ROWS_PT
EPT = E // NT
CH = 32
NCH = EPT // CH
HP = 16
S_ROWS = 1280
SPT = S_ROWS // NT


def _leaky(x):
    return jnp.where(x >= 0, x, NEG * x)


def _dense_body(f_ref, wn_ref, ws_ref, bs_ref, bd_ref,
                h_ref, hs_ref, es_ref, ed_ref):
    f = f_ref[...]
    h = f @ wn_ref[...]
    hs = h @ ws_ref[...]
    h_ref[...] = h
    hs_ref[...] = hs
    es_ref[...] = hs @ bs_ref[...]
    ed_ref[...] = h @ bd_ref[...]


def _dense(feat, W_node, W_src, Bsrc, Bdst):
    nb = 10
    return pl.pallas_call(
        _dense_body,
        out_shape=(
            jax.ShapeDtypeStruct((N, DHH), jnp.float32),
            jax.ShapeDtypeStruct((N, DHH), jnp.float32),
            jax.ShapeDtypeStruct((N, DHH), jnp.float32),
            jax.ShapeDtypeStruct((N, DHH), jnp.float32),
        ),
        grid=(nb,),
        in_specs=[
            pl.BlockSpec((N // nb, D_IN), lambda i: (i, 0)),
            pl.BlockSpec((D_IN, DHH), lambda i: (0, 0)),
            pl.BlockSpec((DHH, DHH), lambda i: (0, 0)),
            pl.BlockSpec((DHH, DHH), lambda i: (0, 0)),
            pl.BlockSpec((DHH, DHH), lambda i: (0, 0)),
        ],
        out_specs=(
            pl.BlockSpec((N // nb, DHH), lambda i: (i, 0)),
            pl.BlockSpec((N // nb, DHH), lambda i: (i, 0)),
            pl.BlockSpec((N // nb, DHH), lambda i: (i, 0)),
            pl.BlockSpec((N // nb, DHH), lambda i: (i, 0)),
        ),
    )(feat, W_node, W_src, Bsrc, Bdst)


def _rel_body(re_ref, wr_ref, wp_ref, bp_ref, rw_ref, rp_ref):
    re = re_ref[...]
    rw_ref[...] = re @ wr_ref[...]
    rp_ref[...] = re @ wp_ref[...] + bp_ref[...]


def _rel(rel_emb, W_rel, W_prop, b_prop):
    return pl.pallas_call(
        _rel_body,
        out_shape=(
            jax.ShapeDtypeStruct((1, HEADS * 2 * HID), jnp.float32),
            jax.ShapeDtypeStruct((1, HEADS * REL_HID), jnp.float32),
        ),
    )(rel_emb.reshape(1, REL_IN), W_rel, W_prop, b_prop.reshape(1, -1))


_mesh = plsc.VectorSubcoreMesh(core_axis_name="c", subcore_axis_name="s")


@functools.partial(
    pl.kernel,
    out_type=(
        jax.ShapeDtypeStruct((2 * NPAD, DHH), jnp.float32),
        jax.ShapeDtypeStruct((2 * S_ROWS, DHH), jnp.float32),
    ),
    mesh=_mesh,
    scratch_types=(
        [pltpu.VMEM((CH,), jnp.int32)] * 12
        + [pltpu.VMEM((CH, DHH), jnp.float32)] * 9
        + [
            pltpu.VMEM((HP,), jnp.float32),
            pltpu.VMEM_SHARED((NPAD, DHH), jnp.float32),
            pltpu.VMEM_SHARED((S_ROWS, DHH), jnp.float32),
        ]
        + [pltpu.SemaphoreType.DMA] * 9
    ),
)
def _sc_edge(es_hbm, ed_hbm, hs_hbm, src_hbm, dstg_hbm, dstl_hbm, shift_hbm,
             z_hbm, out_hbm, s_out_hbm,
             src_v0, src_v1, src_v2, dstg_v0, dstg_v1, dstg_v2,
             dstl_v0, dstl_v1, dstl_v2, sidx_v0, sidx_v1, sidx_v2,
             esx_v0, esx_v1, esx_v2, ed_v0, ed_v1, ed_v2,
             hs_v0, hs_v1, hs_v2, shift_v, out_sh, s_sh,
             ixsem0, ixsem1, ixsem2, gsem0, gsem1, gsem2,
             ssem0, ssem1, ssem2):
    cid = lax.axis_index("c")
    sid = lax.axis_index("s")

    srcs = (src_v0, src_v1, src_v2)
    dstgs = (dstg_v0, dstg_v1, dstg_v2)
    dstls = (dstl_v0, dstl_v1, dstl_v2)
    sidxs = (sidx_v0, sidx_v1, sidx_v2)
    esxs = (esx_v0, esx_v1, esx_v2)
    eds = (ed_v0, ed_v1, ed_v2)
    hss = (hs_v0, hs_v1, hs_v2)
    ixsems = (ixsem0, ixsem1, ixsem2)
    gsems = (gsem0, gsem1, gsem2)
    ssems = (ssem0, ssem1, ssem2)

    pltpu.sync_copy(z_hbm, out_sh.at[pl.ds(sid * ROWS_PT, ROWS_PT)])
    pltpu.sync_copy(z_hbm.at[pl.ds(0, SPT)], s_sh.at[pl.ds(sid * SPT, SPT)])
    pltpu.sync_copy(shift_hbm.at[pl.ds(cid * HP, HP)], shift_v)
    plsc.subcore_barrier()

    ebase = cid * E + sid * EPT
    sh = shift_v[...]
    zv = jnp.zeros((HP,), jnp.float32)

    def issue_idx(ci, b):
        base = ebase + ci * CH
        sem = ixsems[b]
        return (pltpu.async_copy(src_hbm.at[pl.ds(base, CH)], srcs[b], sem),
                pltpu.async_copy(dstg_hbm.at[pl.ds(base, CH)], dstgs[b], sem),
                pltpu.async_copy(dstl_hbm.at[pl.ds(base, CH)], dstls[b], sem))

    def issue_gather(b):
        sem = gsems[b]
        return (pltpu.async_copy(es_hbm.at[srcs[b]], esxs[b], sem),
                pltpu.async_copy(ed_hbm.at[dstgs[b]], eds[b], sem),
                pltpu.async_copy(hs_hbm.at[srcs[b]], hss[b], sem))

    def compute(b, g):
        dstl_v, sidx_v, esx_v, ed_v, hs_v = (
            dstls[b], sidxs[b], esxs[b], eds[b], hss[b])
        g[0].wait()
        g[1].wait()
        g[2].wait()

        def group(gi, carry2):
            dvec = dstl_v[pl.ds(gi * HP, HP)]
            sidx_v[pl.ds(gi * HP, HP)] = lax.shift_right_logical(dvec, 3)
            for k in range(HP):
                e = gi * HP + k
                x = esx_v[e, pl.ds(0, HP)] + ed_v[e, pl.ds(0, HP)]
                x = jnp.where(x >= 0, x, NEG * x)
                ex = jnp.exp(x - sh)
                d = dvec[k]
                for slot in range(8):
                    esx_v[e, pl.ds(slot * HP, HP)] = zv
                esx_v[e, pl.ds((d & 7) * HP, HP)] = ex
                for h in range(HEADS):
                    av = jnp.full((HID,), ex[h], jnp.float32)
                    col = h * HID
                    hs_v[e, pl.ds(col, HID)] = hs_v[e, pl.ds(col, HID)] * av
            return carry2

        lax.fori_loop(0, CH // HP, group, 0)
        sem = ssems[b]
        return (pltpu.async_copy(esx_v, s_sh.at[sidx_v], sem, add=True),
                pltpu.async_copy(hs_v, out_sh.at[dstl_v], sem, add=True))

    def tri(i, carry):
        c0 = 3 * i
        ix0 = issue_idx(c0, 0)
        ix1 = issue_idx(c0 + 1, 1)
        ix2 = issue_idx(c0 + 2, 2)
        for h in ix0:
            h.wait()
        g0 = issue_gather(0)
        for h in ix1:
            h.wait()
        g1 = issue_gather(1)
        for h in ix2:
            h.wait()
        g2 = issue_gather(2)
        s0 = compute(0, g0)
        s1 = compute(1, g1)
        s2 = compute(2, g2)
        for h in s0 + s1 + s2:
            h.wait()
        return carry

    lax.fori_loop(0, NCH // 3, tri, 0)

    ixt = issue_idx(NCH - 1, 0)
    for h in ixt:
        h.wait()
    st = compute(0, issue_gather(0))
    for h in st:
        h.wait()

    plsc.subcore_barrier()
    pltpu.sync_copy(out_sh.at[pl.ds(sid * ROWS_PT, ROWS_PT)],
                    out_hbm.at[pl.ds(cid * NPAD + sid * ROWS_PT, ROWS_PT)])
    pltpu.sync_copy(s_sh.at[pl.ds(sid * SPT, SPT)],
                    s_out_hbm.at[pl.ds(cid * S_ROWS + sid * SPT, SPT)])


def _post_body(om1_ref, om2_ref, s1_ref, s2_ref, f1_ref, f2_ref,
               wres_ref, bres_ref, rw_ref, eh16_ref, eh8_ref,
               bc1_ref, bc2_ref, o1_ref, o2_ref):
    eh16 = eh16_ref[...]
    eh8 = eh8_ref[...]
    alpha = 1.0 / (1.0 + jnp.exp(-rw_ref[0, 0]))

    inv1 = 1.0 / (s1_ref[...] + 1e-16)
    out1 = jnp.maximum(om1_ref[...] * (inv1 @ eh8), 0.0)
    inv2 = 1.0 / (s2_ref[...] + 1e-16)
    out2 = jnp.maximum(om2_ref[...] * (inv2 @ eh8), 0.0)

    res1 = f1_ref[...] @ wres_ref[...] + bres_ref[...]
    res2 = f2_ref[...] @ wres_ref[...] + bres_ref[...]
    out1 = out1 * alpha + res1 * (1.0 - alpha)
    out2 = out2 * alpha + res2 * (1.0 - alpha)

    def cross(bc):
        t1 = _leaky(out1 @ bc)
        t2 = _leaky(out2 @ bc)
        m = jnp.maximum(t1, t2)
        w1 = jnp.exp(t1 - m)
        w2 = jnp.exp(t2 - m)
        sw = w1 + w2
        return out1 * ((w1 / sw) @ eh8) + out2 * ((w2 / sw) @ eh8)

    o1_ref[...] = cross(bc1_ref[...])
    o2_ref[...] = cross(bc2_ref[...])


def _post(om1, om2, s1, s2, feat1, feat2, W_res, b_res, res_w, Eh16, Eh8,
          Bc1, Bc2):
    nb = 10
    blk = N // nb
    return pl.pallas_call(
        _post_body,
        out_shape=(
            jax.ShapeDtypeStruct((N, DHH), jnp.float32),
            jax.ShapeDtypeStruct((N, DHH), jnp.float32),
        ),
        grid=(nb,),
        in_specs=[
            pl.BlockSpec((blk, DHH), lambda i: (i, 0)),
            pl.BlockSpec((blk, DHH), lambda i: (i, 0)),
            pl.BlockSpec((blk, HEADS), lambda i: (i, 0)),
            pl.BlockSpec((blk, HEADS), lambda i: (i, 0)),
            pl.BlockSpec((blk, D_IN), lambda i: (i, 0)),
            pl.BlockSpec((blk, D_IN), lambda i: (i, 0)),
            pl.BlockSpec((D_IN, DHH), lambda i: (0, 0)),
            pl.BlockSpec((1, DHH), lambda i: (0, 0)),
            pl.BlockSpec((1, 1), lambda i: (0, 0)),
            pl.BlockSpec((HP, DHH), lambda i: (0, 0)),
            pl.BlockSpec((HEADS, DHH), lambda i: (0, 0)),
            pl.BlockSpec((DHH, HEADS), lambda i: (0, 0)),
            pl.BlockSpec((DHH, HEADS), lambda i: (0, 0)),
        ],
        out_specs=(
            pl.BlockSpec((blk, DHH), lambda i: (i, 0)),
            pl.BlockSpec((blk, DHH), lambda i: (i, 0)),
        ),
    )(om1, om2, s1, s2, feat1, feat2, W_res, b_res.reshape(1, DHH),
      res_w.reshape(1, 1), Eh16, Eh8, Bc1, Bc2)


def kernel(feat1, feat2, rel_emb1, rel_emb2, edge_index1, edge_index2,
           W_node, W_src1, W_src2, W_rel1, W_rel2,
           W_res, b_res, res_w, cross_w1, cross_w2,
           W_prop1, W_prop2, b_prop1, b_prop2):
    rw1, rp1 = _rel(rel_emb1, W_rel1, W_prop1, b_prop1)
    rw2, rp2 = _rel(rel_emb2, W_rel2, W_prop2, b_prop2)

    m8 = jnp.repeat(jnp.eye(HEADS, dtype=jnp.float32), HID, axis=0)
    m128 = jnp.pad(m8, ((0, 0), (0, DHH - HEADS)))
    rwa = rw1.reshape(HEADS, 2 * HID)
    rwb = rw2.reshape(HEADS, 2 * HID)
    Bdst1 = rwa[:, :HID].reshape(DHH, 1) * m128
    Bsrc1 = rwa[:, HID:].reshape(DHH, 1) * m128
    Bdst2 = rwb[:, :HID].reshape(DHH, 1) * m128
    Bsrc2 = rwb[:, HID:].reshape(DHH, 1) * m128

    h1, hs1, es1, ed1 = _dense(feat1, W_node, W_src1, Bsrc1, Bdst1)
    h2, hs2, es2, ed2 = _dense(feat2, W_node, W_src2, Bsrc2, Bdst2)

    def shift_of(es, ed):
        c = jnp.maximum(jnp.max(es[:, :HEADS], axis=0)
                        + jnp.max(ed[:, :HEADS], axis=0), 0.0)
        return jnp.pad(c, (0, HP - HEADS))

    shift = jnp.concatenate([shift_of(es1, ed1), shift_of(es2, ed2)])

    es_all = jnp.concatenate([es1, es2])
    ed_all = jnp.concatenate([ed1, ed2])
    hs_all = jnp.concatenate([hs1, hs2])
    src_g = jnp.concatenate([edge_index1[0], edge_index2[0] + N])
    dst_g = jnp.concatenate([edge_index1[1], edge_index2[1] + N])
    dst_l = jnp.concatenate([edge_index1[1], edge_index2[1]])
    z = jnp.zeros((ROWS_PT, DHH), jnp.float32)

    out_msg, s_pack = _sc_edge(es_all, ed_all, hs_all, src_g, dst_g, dst_l,
                               shift, z)

    Eh8 = jnp.repeat(jnp.eye(HEADS, dtype=jnp.float32), HID, axis=1)
    Eh16 = jnp.pad(Eh8, ((0, HP - HEADS), (0, 0)))
    Bc1 = cross_w1.reshape(DHH, 1) * m8
    Bc2 = cross_w2.reshape(DHH, 1) * m8

    s_un = s_pack.reshape(2, S_ROWS * 8, HP)[:, :N, :HEADS]
    o1, o2 = _post(out_msg[:N], out_msg[NPAD:NPAD + N],
                   s_un[0], s_un[1],
                   feat1, feat2, W_res, b_res, res_w, Eh16, Eh8, Bc1, Bc2)

    return (o1, o2, rp1.reshape(-1), rp2.reshape(-1),
            h1.reshape(N, HEADS, HID), h2.reshape(N, HEADS, HID))
